# Initial kernel scaffold; baseline (speedup 1.0000x reference)
import jax
import jax.numpy as jnp
from jax.experimental import pallas as pl


def kernel(x, table):
    raise NotImplementedError("write your pallas kernel here")


# speedup vs baseline: 3.3345x; 3.3345x over previous
import functools

import jax
import jax.numpy as jnp
from jax import lax
from jax.experimental import pallas as pl
from jax.experimental.pallas import tpu as pltpu
from jax.experimental.pallas import tpu_sc as plsc

NC = 2
NS = 16
NW = NC * NS
CHUNK = 128


def _make_gather(V, D, B):
    assert B % (NW * CHUNK) == 0
    bpw = B // NW
    nch = bpw // CHUNK
    mesh = plsc.VectorSubcoreMesh(
        core_axis_name="c", subcore_axis_name="s",
        num_cores=NC, num_subcores=NS)

    @functools.partial(
        pl.kernel,
        out_type=jax.ShapeDtypeStruct((B, D), jnp.float32),
        mesh=mesh,
        scratch_types=[
            pltpu.VMEM((nch, CHUNK), jnp.int32),
            pltpu.VMEM((CHUNK, D), jnp.float32),
            pltpu.VMEM((CHUNK, D), jnp.float32),
            pltpu.SemaphoreType.DMA,
            pltpu.SemaphoreType.DMA,
        ],
    )
    def gather_kernel(table_hbm, idx_hbm, out_hbm, idx_v, buf0, buf1,
                      sem0, sem1):
        wid = lax.axis_index("s") * NC + lax.axis_index("c")
        base = wid * bpw
        pltpu.sync_copy(idx_hbm.at[wid], idx_v)

        pltpu.async_copy(table_hbm.at[idx_v.at[0]], buf0, sem0)

        @pl.loop(0, nch // 2)
        def _(g):
            j0 = 2 * g
            pltpu.async_copy(table_hbm.at[idx_v.at[j0 + 1]], buf1, sem1)
            pltpu.make_async_copy(
                table_hbm.at[idx_v.at[j0]], buf0, sem0).wait()
            pltpu.sync_copy(buf0, out_hbm.at[pl.ds(base + j0 * CHUNK, CHUNK)])

            @pl.when(g + 1 < nch // 2)
            def _():
                pltpu.async_copy(table_hbm.at[idx_v.at[j0 + 2]], buf0, sem0)

            pltpu.make_async_copy(
                table_hbm.at[idx_v.at[j0 + 1]], buf1, sem1).wait()
            pltpu.sync_copy(
                buf1, out_hbm.at[pl.ds(base + (j0 + 1) * CHUNK, CHUNK)])

    return gather_kernel


def kernel(x, table):
    V, D = table.shape
    B = x.size
    idx = x.reshape(NW, B // (NW * CHUNK), CHUNK).astype(jnp.int32)
    out = _make_gather(V, D, B)(table, idx)
    return out.reshape(x.shape + (D,))
